# Initial kernel scaffold; baseline (speedup 1.0000x reference)
import jax
import jax.numpy as jnp
from jax.experimental import pallas as pl


def kernel(x, edge_index, edge_attr, params):
    raise NotImplementedError("write your pallas kernel here")


# speedup vs baseline: 22.2866x; 22.2866x over previous
import functools

import jax
import jax.numpy as jnp
import numpy as np
from jax import lax
from jax.experimental import pallas as pl
from jax.experimental.pallas import tpu as pltpu
from jax.experimental.pallas import tpu_sc as plsc

N = 10000
E = 160000
D = 128
H = 8
DH = 16
EIN = 16
NC = 7
EC = 2

SC_CORES = 2
SC_TILES = 16
NWORKERS = SC_CORES * SC_TILES
CHUNK = 128

BN_ROWS = 2000
BE_ROWS = 4000

_f32 = jnp.float32


def _head_sel_128x8(dtype=_f32):
    d = lax.broadcasted_iota(jnp.int32, (D, H), 0)
    h = lax.broadcasted_iota(jnp.int32, (D, H), 1)
    return (d // DH == h).astype(dtype)


def _head_sel_8x128(dtype=_f32):
    h = lax.broadcasted_iota(jnp.int32, (H, D), 0)
    d = lax.broadcasted_iota(jnp.int32, (H, D), 1)
    return (d // DH == h).astype(dtype)


def _head_sel_16x128(dtype=_f32):
    j = lax.broadcasted_iota(jnp.int32, (16, D), 0)
    d = lax.broadcasted_iota(jnp.int32, (16, D), 1)
    return ((d // DH == j) & (j < H)).astype(dtype)


def _head_sel_8x16(dtype=_f32):
    h = lax.broadcasted_iota(jnp.int32, (H, 16), 0)
    j = lax.broadcasted_iota(jnp.int32, (H, 16), 1)
    return (j == h).astype(dtype)


def _dot(a, b):
    return jnp.dot(a, b, preferred_element_type=_f32,
                   precision=lax.Precision.HIGHEST)


def _affine_mm_body(x_ref, sc_ref, sh_ref, w_ref, b_ref, o_ref):
    xn = x_ref[...] * sc_ref[...] + sh_ref[...]
    o_ref[...] = _dot(xn, w_ref[...]) + b_ref[...]


def _affine_mm(x, scale, shift, w, b, block_rows):
    rows, din = x.shape
    dout = w.shape[1]
    grid = rows // block_rows
    return pl.pallas_call(
        _affine_mm_body,
        grid=(grid,),
        in_specs=[
            pl.BlockSpec((block_rows, din), lambda i: (i, 0)),
            pl.BlockSpec((1, din), lambda i: (0, 0)),
            pl.BlockSpec((1, din), lambda i: (0, 0)),
            pl.BlockSpec((din, dout), lambda i: (0, 0)),
            pl.BlockSpec((1, dout), lambda i: (0, 0)),
        ],
        out_specs=pl.BlockSpec((block_rows, dout), lambda i: (i, 0)),
        out_shape=jax.ShapeDtypeStruct((rows, dout), _f32),
    )(x, scale.reshape(1, din), shift.reshape(1, din), w, b.reshape(1, dout))


def _kvq_body(x_ref, sc_ref, sh_ref, wkv_ref, wq_ref, kv_ref, q_ref):
    xn = x_ref[...] * sc_ref[...] + sh_ref[...]
    kv_ref[...] = _dot(xn, wkv_ref[...])
    q_ref[...] = _dot(xn, wq_ref[...])


def _kvq(x, scale, shift, wkv, wq):
    grid = N // BN_ROWS
    return pl.pallas_call(
        _kvq_body,
        grid=(grid,),
        in_specs=[
            pl.BlockSpec((BN_ROWS, D), lambda i: (i, 0)),
            pl.BlockSpec((1, D), lambda i: (0, 0)),
            pl.BlockSpec((1, D), lambda i: (0, 0)),
            pl.BlockSpec((D, 2 * D), lambda i: (0, 0)),
            pl.BlockSpec((D, D), lambda i: (0, 0)),
        ],
        out_specs=[
            pl.BlockSpec((BN_ROWS, 2 * D), lambda i: (i, 0)),
            pl.BlockSpec((BN_ROWS, D), lambda i: (i, 0)),
        ],
        out_shape=[
            jax.ShapeDtypeStruct((N, 2 * D), _f32),
            jax.ShapeDtypeStruct((N, D), _f32),
        ],
    )(x, scale.reshape(1, D), shift.reshape(1, D), wkv, wq)


def _score_body(gkv_ref, gq_ref, ep_ref, er_ref, esc_ref, esh_ref, woe_ref,
                ye_ref, pa_ref, pb_ref, st_ref):
    i = pl.program_id(0)
    kv = gkv_ref[...]
    ksrc = kv[:, :D]
    vsrc = kv[:, D:]
    score = ksrc * gq_ref[...] * (0.25 * ep_ref[...])
    er = er_ref[...] * esc_ref[...] + esh_ref[...]
    ye = _dot(score, woe_ref[...]) + er
    ye_ref[...] = ye
    logit = jnp.clip(_dot(score, _head_sel_128x8()), -5.0, 5.0)
    ex = jnp.exp(logit)
    ex128 = _dot(ex, _head_sel_8x128())
    pa_ref[...] = ex128 * vsrc
    pb_ref[...] = ex128
    s0 = jnp.sum(ye, axis=0)[None, :]
    s1 = jnp.sum(ye * ye, axis=0)[None, :]
    st = jnp.concatenate([s0, s1, jnp.zeros((6, D), _f32)], axis=0)

    @pl.when(i == 0)
    def _():
        st_ref[...] = st

    @pl.when(i > 0)
    def _():
        st_ref[...] = st_ref[...] + st


def _score(gkv, gq, ep, eres, esc, esh, woe):
    grid = E // BE_ROWS
    return pl.pallas_call(
        _score_body,
        grid=(grid,),
        in_specs=[
            pl.BlockSpec((BE_ROWS, 2 * D), lambda i: (i, 0)),
            pl.BlockSpec((BE_ROWS, D), lambda i: (i, 0)),
            pl.BlockSpec((BE_ROWS, D), lambda i: (i, 0)),
            pl.BlockSpec((BE_ROWS, D), lambda i: (i, 0)),
            pl.BlockSpec((1, D), lambda i: (0, 0)),
            pl.BlockSpec((1, D), lambda i: (0, 0)),
            pl.BlockSpec((D, D), lambda i: (0, 0)),
        ],
        out_specs=[
            pl.BlockSpec((BE_ROWS, D), lambda i: (i, 0)),
            pl.BlockSpec((BE_ROWS, D), lambda i: (i, 0)),
            pl.BlockSpec((BE_ROWS, D), lambda i: (i, 0)),
            pl.BlockSpec((8, D), lambda i: (0, 0)),
        ],
        out_shape=[
            jax.ShapeDtypeStruct((E, D), _f32),
            jax.ShapeDtypeStruct((E, D), _f32),
            jax.ShapeDtypeStruct((E, D), _f32),
            jax.ShapeDtypeStruct((8, D), _f32),
        ],
    )(gkv, gq, ep, eres, esc.reshape(1, D), esh.reshape(1, D), woe)


def _nodeup_body(a0_ref, a1_ref, x_ref, xsc_ref, xsh_ref,
                 won_ref, y_ref, st_ref):
    i = pl.program_id(0)
    agg = a0_ref[0] / (a1_ref[0] + 1e-16)
    nout = _dot(agg, won_ref[...])
    y = nout + x_ref[...] * xsc_ref[...] + xsh_ref[...]
    y_ref[...] = y
    s0 = jnp.sum(y, axis=0)[None, :]
    s1 = jnp.sum(y * y, axis=0)[None, :]
    st = jnp.concatenate([s0, s1, jnp.zeros((6, D), _f32)], axis=0)

    @pl.when(i == 0)
    def _():
        st_ref[...] = st

    @pl.when(i > 0)
    def _():
        st_ref[...] = st_ref[...] + st


def _nodeup(acc, x, xsc, xsh, won):
    grid = N // BN_ROWS
    return pl.pallas_call(
        _nodeup_body,
        grid=(grid,),
        in_specs=[
            pl.BlockSpec((1, BN_ROWS, D), lambda i: (0, i, 0)),
            pl.BlockSpec((1, BN_ROWS, D), lambda i: (1, i, 0)),
            pl.BlockSpec((BN_ROWS, D), lambda i: (i, 0)),
            pl.BlockSpec((1, D), lambda i: (0, 0)),
            pl.BlockSpec((1, D), lambda i: (0, 0)),
            pl.BlockSpec((D, D), lambda i: (0, 0)),
        ],
        out_specs=[
            pl.BlockSpec((BN_ROWS, D), lambda i: (i, 0)),
            pl.BlockSpec((8, D), lambda i: (0, 0)),
        ],
        out_shape=[
            jax.ShapeDtypeStruct((N, D), _f32),
            jax.ShapeDtypeStruct((8, D), _f32),
        ],
    )(acc, acc, x, xsc.reshape(1, D), xsh.reshape(1, D), won)


def _ffn_body(t_ref, sc_ref, sh_ref, w1_ref, b1_ref, w2_ref, b2_ref,
              z_ref, st_ref):
    i = pl.program_id(0)
    xb = t_ref[...] * sc_ref[...] + sh_ref[...]
    h = jnp.maximum(_dot(xb, w1_ref[...]) + b1_ref[...], 0.0)
    z = xb + _dot(h, w2_ref[...]) + b2_ref[...]
    z_ref[...] = z
    s0 = jnp.sum(z, axis=0)[None, :]
    s1 = jnp.sum(z * z, axis=0)[None, :]
    st = jnp.concatenate([s0, s1, jnp.zeros((6, D), _f32)], axis=0)

    @pl.when(i == 0)
    def _():
        st_ref[...] = st

    @pl.when(i > 0)
    def _():
        st_ref[...] = st_ref[...] + st


def _ffn(t, scale, shift, w1, b1, w2, b2, block_rows):
    rows = t.shape[0]
    hid = w1.shape[1]
    grid = rows // block_rows
    return pl.pallas_call(
        _ffn_body,
        grid=(grid,),
        in_specs=[
            pl.BlockSpec((block_rows, D), lambda i: (i, 0)),
            pl.BlockSpec((1, D), lambda i: (0, 0)),
            pl.BlockSpec((1, D), lambda i: (0, 0)),
            pl.BlockSpec((D, hid), lambda i: (0, 0)),
            pl.BlockSpec((1, hid), lambda i: (0, 0)),
            pl.BlockSpec((hid, D), lambda i: (0, 0)),
            pl.BlockSpec((1, D), lambda i: (0, 0)),
        ],
        out_specs=[
            pl.BlockSpec((block_rows, D), lambda i: (i, 0)),
            pl.BlockSpec((8, D), lambda i: (0, 0)),
        ],
        out_shape=[
            jax.ShapeDtypeStruct((rows, D), _f32),
            jax.ShapeDtypeStruct((8, D), _f32),
        ],
    )(t, scale.reshape(1, D), shift.reshape(1, D), w1, b1.reshape(1, hid),
      w2, b2.reshape(1, D))


_TOTAL_CHUNKS = E // CHUNK
_BASE_CHUNKS = _TOTAL_CHUNKS // NWORKERS
_EXTRA = _TOTAL_CHUNKS - _BASE_CHUNKS * NWORKERS

_TILE_BASE = _TOTAL_CHUNKS // SC_TILES
_TILE_EXTRA = _TOTAL_CHUNKS - _TILE_BASE * SC_TILES
NPAD = 10240
_NPC = NPAD // SC_TILES


def _sc_gather(kv, q, src, dst):
    mesh = plsc.VectorSubcoreMesh(core_axis_name="c", subcore_axis_name="s")

    @functools.partial(
        pl.kernel,
        out_type=(
            jax.ShapeDtypeStruct((E, 2 * D), _f32),
            jax.ShapeDtypeStruct((E, D), _f32),
        ),
        mesh=mesh,
        scratch_types=[
            pltpu.VMEM((CHUNK,), jnp.int32),
            pltpu.VMEM((CHUNK,), jnp.int32),
            pltpu.VMEM((CHUNK, 2 * D), _f32),
            pltpu.VMEM((CHUNK, D), _f32),
            pltpu.SemaphoreType.DMA,
            pltpu.SemaphoreType.DMA,
        ],
    )
    def k(kv_hbm, q_hbm, src_hbm, dst_hbm, okv_hbm, oq_hbm,
          src_v, dst_v, kv_b, q_b, sem1, sem2):
        wid = lax.axis_index("s") * SC_CORES + lax.axis_index("c")
        nchunks = _BASE_CHUNKS + jnp.where(wid < _EXTRA, 1, 0)

        def body(i, carry):
            base = (wid + i * NWORKERS) * CHUNK
            pltpu.sync_copy(src_hbm.at[pl.ds(base, CHUNK)], src_v)
            pltpu.sync_copy(dst_hbm.at[pl.ds(base, CHUNK)], dst_v)
            c1 = pltpu.async_copy(kv_hbm.at[src_v], kv_b, sem1)
            c2 = pltpu.async_copy(q_hbm.at[dst_v], q_b, sem2)
            c1.wait()
            c2.wait()
            pltpu.sync_copy(kv_b, okv_hbm.at[pl.ds(base, CHUNK)])
            pltpu.sync_copy(q_b, oq_hbm.at[pl.ds(base, CHUNK)])
            return carry

        lax.fori_loop(0, nchunks, body, 0)

    return k(kv, q, src, dst)


def _sc_scatter(pay_a, pay_b, dst, zeros_a):
    mesh = plsc.VectorSubcoreMesh(core_axis_name="c", subcore_axis_name="s")

    @functools.partial(
        pl.kernel,
        out_type=jax.ShapeDtypeStruct((2, NPAD, D), _f32),
        mesh=mesh,
        scratch_types=[
            pltpu.VMEM((CHUNK,), jnp.int32),
            pltpu.VMEM((CHUNK, D), _f32),
            pltpu.VMEM_SHARED((NPAD, D), _f32),
        ],
    )
    def k(pa_hbm, pb_hbm, dst_hbm, za_hbm, oa_hbm, idx_v, p_v, sh_a):
        cid = lax.axis_index("c")
        sid = lax.axis_index("s")
        row0 = sid * _NPC
        pltpu.sync_copy(za_hbm.at[pl.ds(row0, _NPC)],
                        sh_a.at[pl.ds(row0, _NPC)])
        plsc.subcore_barrier()

        nchunks = _TILE_BASE + jnp.where(sid < _TILE_EXTRA, 1, 0)

        def make_body(src_hbm):
            def body(j, carry):
                base = (sid + j * SC_TILES) * CHUNK
                pltpu.sync_copy(dst_hbm.at[pl.ds(base, CHUNK)], idx_v)
                pltpu.sync_copy(src_hbm.at[pl.ds(base, CHUNK)], p_v)
                pltpu.sync_copy(p_v, sh_a.at[idx_v], add=True)
                return carry
            return body

        @pl.when(cid == 0)
        def _():
            lax.fori_loop(0, nchunks, make_body(pa_hbm), 0)

        @pl.when(cid == 1)
        def _():
            lax.fori_loop(0, nchunks, make_body(pb_hbm), 0)

        plsc.subcore_barrier()
        pltpu.sync_copy(sh_a.at[pl.ds(row0, _NPC)],
                        oa_hbm.at[cid, pl.ds(row0, _NPC)])

    return k(pay_a, pay_b, dst, zeros_a)


def _bn_affine(st, rows, g, b):
    s0 = st[0]
    s1 = st[1]
    mean = s0 / rows
    var = s1 / rows - mean * mean
    rstd = lax.rsqrt(var + 1e-5)
    scale = g * rstd
    shift = b - mean * scale
    return scale, shift


def kernel(x, edge_index, edge_attr, params):
    src = edge_index[0]
    dst = edge_index[1]
    e_pad = jnp.pad(edge_attr, ((0, 0), (0, D - EIN)))

    ones = jnp.ones((D,), _f32)
    zeros = jnp.zeros((D,), _f32)
    zeros_a = jnp.zeros((NPAD, D), _f32)

    t_n, sc_n, sh_n = x, ones, zeros
    t_e, sc_e, sh_e = e_pad, ones, zeros

    for i, p in enumerate(params['layers']):
        wkv = jnp.concatenate([p['Wk'], p['Wv']], axis=1)
        kv, q = _kvq(t_n, sc_n, sh_n, wkv, p['Wq'])

        if i == 0:
            we = jnp.pad(p['We'], ((0, D - EIN), (0, 0)))
            ep = _affine_mm(t_e, sc_e, sh_e, we, zeros, BE_ROWS)
            resw = jnp.pad(p['res_e_W'], ((0, D - EIN), (0, 0)))
            eres = _affine_mm(t_e, sc_e, sh_e, resw, p['res_e_b'], BE_ROWS)
            er_t, er_sc, er_sh = eres, ones, zeros
        else:
            ep = _affine_mm(t_e, sc_e, sh_e, p['We'], zeros, BE_ROWS)
            er_t, er_sc, er_sh = t_e, sc_e, sh_e

        gkv, gq = _sc_gather(kv, q, src, dst)
        ye, pa, pb, st_e = _score(gkv, gq, ep, er_t, er_sc, er_sh,
                                  p['Wo_e'])
        acc = _sc_scatter(pa, pb, dst, zeros_a)
        y_n, st_n = _nodeup(acc, t_n, sc_n, sh_n, p['Wo_n'])

        sc1_n, sh1_n = _bn_affine(st_n, N, p['nx_g'], p['nx_b'])
        sc1_e, sh1_e = _bn_affine(st_e, E, p['ne_g'], p['ne_b'])

        z_n, stz_n = _ffn(y_n, sc1_n, sh1_n, p['fn_W1'], p['fn_b1'],
                          p['fn_W2'], p['fn_b2'], BN_ROWS)
        z_e, stz_e = _ffn(ye, sc1_e, sh1_e, p['fe_W1'], p['fe_b1'],
                          p['fe_W2'], p['fe_b2'], BE_ROWS)

        sc_n, sh_n = _bn_affine(stz_n, N, p['fn_g'], p['fn_b'])
        sc_e, sh_e = _bn_affine(stz_e, E, p['fe_g'], p['fe_b'])
        t_n, t_e = z_n, z_e

    npw = jnp.pad(params['np_W'], ((0, 0), (0, D - NC)))
    npb = jnp.pad(params['np_b'], ((0, D - NC),))
    epw = jnp.pad(params['ep_W'], ((0, 0), (0, D - EC)))
    epb = jnp.pad(params['ep_b'], ((0, D - EC),))
    node_pred = _affine_mm(t_n, sc_n, sh_n, npw, npb, BN_ROWS)[:, :NC]
    edge_pred = _affine_mm(t_e, sc_e, sh_e, epw, epb, BE_ROWS)[:, :EC]
    return node_pred, edge_pred, x


# speedup vs baseline: 24.1306x; 1.0827x over previous
import functools

import jax
import jax.numpy as jnp
import numpy as np
from jax import lax
from jax.experimental import pallas as pl
from jax.experimental.pallas import tpu as pltpu
from jax.experimental.pallas import tpu_sc as plsc

N = 10000
E = 160000
D = 128
H = 8
DH = 16
EIN = 16
NC = 7
EC = 2

SC_CORES = 2
SC_TILES = 16
NWORKERS = SC_CORES * SC_TILES
CHUNK = 128

BN_ROWS = 2000
BE_ROWS = 4000

_f32 = jnp.float32


def _head_sel_128x8(dtype=_f32):
    d = lax.broadcasted_iota(jnp.int32, (D, H), 0)
    h = lax.broadcasted_iota(jnp.int32, (D, H), 1)
    return (d // DH == h).astype(dtype)


def _head_sel_8x128(dtype=_f32):
    h = lax.broadcasted_iota(jnp.int32, (H, D), 0)
    d = lax.broadcasted_iota(jnp.int32, (H, D), 1)
    return (d // DH == h).astype(dtype)


def _head_sel_16x128(dtype=_f32):
    j = lax.broadcasted_iota(jnp.int32, (16, D), 0)
    d = lax.broadcasted_iota(jnp.int32, (16, D), 1)
    return ((d // DH == j) & (j < H)).astype(dtype)


def _head_sel_8x16(dtype=_f32):
    h = lax.broadcasted_iota(jnp.int32, (H, 16), 0)
    j = lax.broadcasted_iota(jnp.int32, (H, 16), 1)
    return (j == h).astype(dtype)


def _dot(a, b):
    return jnp.dot(a, b, preferred_element_type=_f32,
                   precision=lax.Precision.HIGHEST)


def _affine_mm_body(x_ref, sc_ref, sh_ref, w_ref, b_ref, o_ref):
    xn = x_ref[...] * sc_ref[...] + sh_ref[...]
    o_ref[...] = _dot(xn, w_ref[...]) + b_ref[...]


def _affine_mm(x, scale, shift, w, b, block_rows):
    rows, din = x.shape
    dout = w.shape[1]
    grid = rows // block_rows
    return pl.pallas_call(
        _affine_mm_body,
        grid=(grid,),
        in_specs=[
            pl.BlockSpec((block_rows, din), lambda i: (i, 0)),
            pl.BlockSpec((1, din), lambda i: (0, 0)),
            pl.BlockSpec((1, din), lambda i: (0, 0)),
            pl.BlockSpec((din, dout), lambda i: (0, 0)),
            pl.BlockSpec((1, dout), lambda i: (0, 0)),
        ],
        out_specs=pl.BlockSpec((block_rows, dout), lambda i: (i, 0)),
        out_shape=jax.ShapeDtypeStruct((rows, dout), _f32),
    )(x, scale.reshape(1, din), shift.reshape(1, din), w, b.reshape(1, dout))


def _kvq_body(x_ref, sc_ref, sh_ref, wkv_ref, wq_ref, kv_ref, q_ref):
    xn = x_ref[...] * sc_ref[...] + sh_ref[...]
    kv_ref[...] = _dot(xn, wkv_ref[...])
    q_ref[...] = _dot(xn, wq_ref[...])


def _kvq(x, scale, shift, wkv, wq):
    grid = N // BN_ROWS
    return pl.pallas_call(
        _kvq_body,
        grid=(grid,),
        in_specs=[
            pl.BlockSpec((BN_ROWS, D), lambda i: (i, 0)),
            pl.BlockSpec((1, D), lambda i: (0, 0)),
            pl.BlockSpec((1, D), lambda i: (0, 0)),
            pl.BlockSpec((D, 2 * D), lambda i: (0, 0)),
            pl.BlockSpec((D, D), lambda i: (0, 0)),
        ],
        out_specs=[
            pl.BlockSpec((BN_ROWS, 2 * D), lambda i: (i, 0)),
            pl.BlockSpec((BN_ROWS, D), lambda i: (i, 0)),
        ],
        out_shape=[
            jax.ShapeDtypeStruct((N, 2 * D), _f32),
            jax.ShapeDtypeStruct((N, D), _f32),
        ],
    )(x, scale.reshape(1, D), shift.reshape(1, D), wkv, wq)


def _score_body(gkv_ref, gq_ref, ep_ref, er_ref, esc_ref, esh_ref, woe_ref,
                ye_ref, pa_ref, pb_ref, st_ref):
    i = pl.program_id(0)
    kv = gkv_ref[...]
    ksrc = kv[:, :D]
    vsrc = kv[:, D:]
    score = ksrc * gq_ref[...] * (0.25 * ep_ref[...])
    er = er_ref[...] * esc_ref[...] + esh_ref[...]
    ye = _dot(score, woe_ref[...]) + er
    ye_ref[...] = ye
    logit = jnp.clip(_dot(score, _head_sel_128x8()), -5.0, 5.0)
    ex = jnp.exp(logit)
    ex128 = _dot(ex, _head_sel_8x128())
    pa_ref[...] = ex128 * vsrc
    pb_ref[...] = ex128
    s0 = jnp.sum(ye, axis=0)[None, :]
    s1 = jnp.sum(ye * ye, axis=0)[None, :]
    st = jnp.concatenate([s0, s1, jnp.zeros((6, D), _f32)], axis=0)

    @pl.when(i == 0)
    def _():
        st_ref[...] = st

    @pl.when(i > 0)
    def _():
        st_ref[...] = st_ref[...] + st


def _score(gkv, gq, ep, eres, esc, esh, woe):
    grid = E // BE_ROWS
    return pl.pallas_call(
        _score_body,
        grid=(grid,),
        in_specs=[
            pl.BlockSpec((BE_ROWS, 2 * D), lambda i: (i, 0)),
            pl.BlockSpec((BE_ROWS, D), lambda i: (i, 0)),
            pl.BlockSpec((BE_ROWS, D), lambda i: (i, 0)),
            pl.BlockSpec((BE_ROWS, D), lambda i: (i, 0)),
            pl.BlockSpec((1, D), lambda i: (0, 0)),
            pl.BlockSpec((1, D), lambda i: (0, 0)),
            pl.BlockSpec((D, D), lambda i: (0, 0)),
        ],
        out_specs=[
            pl.BlockSpec((BE_ROWS, D), lambda i: (i, 0)),
            pl.BlockSpec((BE_ROWS, D), lambda i: (i, 0)),
            pl.BlockSpec((BE_ROWS, D), lambda i: (i, 0)),
            pl.BlockSpec((8, D), lambda i: (0, 0)),
        ],
        out_shape=[
            jax.ShapeDtypeStruct((E, D), _f32),
            jax.ShapeDtypeStruct((E, D), _f32),
            jax.ShapeDtypeStruct((E, D), _f32),
            jax.ShapeDtypeStruct((8, D), _f32),
        ],
    )(gkv, gq, ep, eres, esc.reshape(1, D), esh.reshape(1, D), woe)


def _nodeup_body(a0_ref, a1_ref, x_ref, xsc_ref, xsh_ref,
                 won_ref, y_ref, st_ref):
    i = pl.program_id(0)
    agg = a0_ref[0] / (a1_ref[0] + 1e-16)
    nout = _dot(agg, won_ref[...])
    y = nout + x_ref[...] * xsc_ref[...] + xsh_ref[...]
    y_ref[...] = y
    s0 = jnp.sum(y, axis=0)[None, :]
    s1 = jnp.sum(y * y, axis=0)[None, :]
    st = jnp.concatenate([s0, s1, jnp.zeros((6, D), _f32)], axis=0)

    @pl.when(i == 0)
    def _():
        st_ref[...] = st

    @pl.when(i > 0)
    def _():
        st_ref[...] = st_ref[...] + st


def _nodeup(acc, x, xsc, xsh, won):
    grid = N // BN_ROWS
    return pl.pallas_call(
        _nodeup_body,
        grid=(grid,),
        in_specs=[
            pl.BlockSpec((1, BN_ROWS, D), lambda i: (0, i, 0)),
            pl.BlockSpec((1, BN_ROWS, D), lambda i: (1, i, 0)),
            pl.BlockSpec((BN_ROWS, D), lambda i: (i, 0)),
            pl.BlockSpec((1, D), lambda i: (0, 0)),
            pl.BlockSpec((1, D), lambda i: (0, 0)),
            pl.BlockSpec((D, D), lambda i: (0, 0)),
        ],
        out_specs=[
            pl.BlockSpec((BN_ROWS, D), lambda i: (i, 0)),
            pl.BlockSpec((8, D), lambda i: (0, 0)),
        ],
        out_shape=[
            jax.ShapeDtypeStruct((N, D), _f32),
            jax.ShapeDtypeStruct((8, D), _f32),
        ],
    )(acc, acc, x, xsc.reshape(1, D), xsh.reshape(1, D), won)


def _ffn_body(t_ref, sc_ref, sh_ref, w1_ref, b1_ref, w2_ref, b2_ref,
              z_ref, st_ref):
    i = pl.program_id(0)
    xb = t_ref[...] * sc_ref[...] + sh_ref[...]
    h = jnp.maximum(_dot(xb, w1_ref[...]) + b1_ref[...], 0.0)
    z = xb + _dot(h, w2_ref[...]) + b2_ref[...]
    z_ref[...] = z
    s0 = jnp.sum(z, axis=0)[None, :]
    s1 = jnp.sum(z * z, axis=0)[None, :]
    st = jnp.concatenate([s0, s1, jnp.zeros((6, D), _f32)], axis=0)

    @pl.when(i == 0)
    def _():
        st_ref[...] = st

    @pl.when(i > 0)
    def _():
        st_ref[...] = st_ref[...] + st


def _ffn(t, scale, shift, w1, b1, w2, b2, block_rows):
    rows = t.shape[0]
    hid = w1.shape[1]
    grid = rows // block_rows
    return pl.pallas_call(
        _ffn_body,
        grid=(grid,),
        in_specs=[
            pl.BlockSpec((block_rows, D), lambda i: (i, 0)),
            pl.BlockSpec((1, D), lambda i: (0, 0)),
            pl.BlockSpec((1, D), lambda i: (0, 0)),
            pl.BlockSpec((D, hid), lambda i: (0, 0)),
            pl.BlockSpec((1, hid), lambda i: (0, 0)),
            pl.BlockSpec((hid, D), lambda i: (0, 0)),
            pl.BlockSpec((1, D), lambda i: (0, 0)),
        ],
        out_specs=[
            pl.BlockSpec((block_rows, D), lambda i: (i, 0)),
            pl.BlockSpec((8, D), lambda i: (0, 0)),
        ],
        out_shape=[
            jax.ShapeDtypeStruct((rows, D), _f32),
            jax.ShapeDtypeStruct((8, D), _f32),
        ],
    )(t, scale.reshape(1, D), shift.reshape(1, D), w1, b1.reshape(1, hid),
      w2, b2.reshape(1, D))


_TOTAL_CHUNKS = E // CHUNK
_BASE_CHUNKS = _TOTAL_CHUNKS // NWORKERS
_EXTRA = _TOTAL_CHUNKS - _BASE_CHUNKS * NWORKERS

_TILE_BASE = _TOTAL_CHUNKS // SC_TILES
_TILE_EXTRA = _TOTAL_CHUNKS - _TILE_BASE * SC_TILES
NPAD = 10240
_NPC = NPAD // SC_TILES


_EPW = E // NWORKERS
_GNF = _EPW // CHUNK
_GTAIL = _EPW - _GNF * CHUNK


def _sc_gather(kv, q, src, dst):
    mesh = plsc.VectorSubcoreMesh(core_axis_name="c", subcore_axis_name="s")

    @functools.partial(
        pl.kernel,
        out_type=(
            jax.ShapeDtypeStruct((E, 2 * D), _f32),
            jax.ShapeDtypeStruct((E, D), _f32),
        ),
        mesh=mesh,
        scratch_types=[
            pltpu.VMEM((_EPW,), jnp.int32),
            pltpu.VMEM((_EPW,), jnp.int32),
            pltpu.VMEM((CHUNK, 2 * D), _f32),
            pltpu.VMEM((CHUNK, 2 * D), _f32),
            pltpu.VMEM((CHUNK, D), _f32),
            pltpu.VMEM((CHUNK, D), _f32),
            pltpu.SemaphoreType.DMA,
            pltpu.SemaphoreType.DMA,
            pltpu.SemaphoreType.DMA,
            pltpu.SemaphoreType.DMA,
        ],
    )
    def k(kv_hbm, q_hbm, src_hbm, dst_hbm, okv, oq,
          srcv, dstv, kva, kvb, qa, qb, g0, g1, w0, w1):
        wid = lax.axis_index("s") * SC_CORES + lax.axis_index("c")
        base = wid * _EPW
        pltpu.sync_copy(src_hbm.at[pl.ds(base, _EPW)], srcv)
        pltpu.sync_copy(dst_hbm.at[pl.ds(base, _EPW)], dstv)
        kvs = (kva, kvb)
        qs = (qa, qb)
        gs = (g0, g1)
        ws = (w0, w1)

        def issue(c, b):
            off = c * CHUNK
            pltpu.async_copy(kv_hbm.at[srcv.at[pl.ds(off, CHUNK)]],
                             kvs[b], gs[b])
            pltpu.async_copy(q_hbm.at[dstv.at[pl.ds(off, CHUNK)]],
                             qs[b], gs[b])

        def wait_g(b):
            pltpu.make_async_copy(kv_hbm.at[srcv.at[pl.ds(0, CHUNK)]],
                                  kvs[b], gs[b]).wait()
            pltpu.make_async_copy(q_hbm.at[dstv.at[pl.ds(0, CHUNK)]],
                                  qs[b], gs[b]).wait()

        def wb(c, b):
            off = base + c * CHUNK
            pltpu.async_copy(kvs[b], okv.at[pl.ds(off, CHUNK)], ws[b])
            pltpu.async_copy(qs[b], oq.at[pl.ds(off, CHUNK)], ws[b])

        def wait_w(b):
            pltpu.make_async_copy(kvs[b], okv.at[pl.ds(base, CHUNK)],
                                  ws[b]).wait()
            pltpu.make_async_copy(qs[b], oq.at[pl.ds(base, CHUNK)],
                                  ws[b]).wait()

        issue(0, 0)

        def body(kk, carry):
            c1 = 2 * kk + 1

            @pl.when(kk > 0)
            def _():
                wait_w(1)

            issue(c1, 1)
            wait_g(0)
            wb(c1 - 1, 0)
            wait_w(0)
            issue(c1 + 1, 0)
            wait_g(1)
            wb(c1, 1)
            return carry

        lax.fori_loop(0, (_GNF - 1) // 2, body, 0)
        wait_g(0)
        wb(_GNF - 1, 0)
        wait_w(1)
        toff = _GNF * CHUNK
        pltpu.async_copy(kv_hbm.at[srcv.at[pl.ds(toff, _GTAIL)]],
                         kvb.at[pl.ds(0, _GTAIL)], g1)
        pltpu.async_copy(q_hbm.at[dstv.at[pl.ds(toff, _GTAIL)]],
                         qb.at[pl.ds(0, _GTAIL)], g1)
        pltpu.make_async_copy(kv_hbm.at[srcv.at[pl.ds(toff, _GTAIL)]],
                              kvb.at[pl.ds(0, _GTAIL)], g1).wait()
        pltpu.make_async_copy(q_hbm.at[dstv.at[pl.ds(toff, _GTAIL)]],
                              qb.at[pl.ds(0, _GTAIL)], g1).wait()
        pltpu.async_copy(kvb.at[pl.ds(0, _GTAIL)],
                         okv.at[pl.ds(base + toff, _GTAIL)], w1)
        pltpu.async_copy(qb.at[pl.ds(0, _GTAIL)],
                         oq.at[pl.ds(base + toff, _GTAIL)], w1)
        wait_w(0)
        pltpu.make_async_copy(kvb.at[pl.ds(0, _GTAIL)],
                              okv.at[pl.ds(base, _GTAIL)], w1).wait()
        pltpu.make_async_copy(qb.at[pl.ds(0, _GTAIL)],
                              oq.at[pl.ds(base, _GTAIL)], w1).wait()

    return k(kv, q, src, dst)


def _sc_scatter(pay_a, pay_b, dst, zeros_a):
    mesh = plsc.VectorSubcoreMesh(core_axis_name="c", subcore_axis_name="s")

    @functools.partial(
        pl.kernel,
        out_type=jax.ShapeDtypeStruct((2, NPAD, D), _f32),
        mesh=mesh,
        scratch_types=[
            pltpu.VMEM((CHUNK,), jnp.int32),
            pltpu.VMEM((CHUNK,), jnp.int32),
            pltpu.VMEM((CHUNK, D), _f32),
            pltpu.VMEM((CHUNK, D), _f32),
            pltpu.VMEM_SHARED((NPAD, D), _f32),
            pltpu.SemaphoreType.DMA,
            pltpu.SemaphoreType.DMA,
            pltpu.SemaphoreType.DMA,
            pltpu.SemaphoreType.DMA,
        ],
    )
    def k(pa_hbm, pb_hbm, dst_hbm, za_hbm, oa_hbm,
          ia, ib, va, vb, sh_a, l0, l1, a0, a1):
        cid = lax.axis_index("c")
        sid = lax.axis_index("s")
        row0 = sid * _NPC
        pltpu.sync_copy(za_hbm.at[pl.ds(row0, _NPC)],
                        sh_a.at[pl.ds(row0, _NPC)])
        plsc.subcore_barrier()

        idxs = (ia, ib)
        pvs = (va, vb)
        ls = (l0, l1)
        asems = (a0, a1)

        def run(src_hbm):
            def ch(kk):
                return sid + kk * SC_TILES

            def issue_load(c, b):
                off = c * CHUNK
                pltpu.async_copy(dst_hbm.at[pl.ds(off, CHUNK)],
                                 idxs[b], ls[b])
                pltpu.async_copy(src_hbm.at[pl.ds(off, CHUNK)],
                                 pvs[b], ls[b])

            def wait_load(b):
                pltpu.make_async_copy(dst_hbm.at[pl.ds(0, CHUNK)],
                                      idxs[b], ls[b]).wait()
                pltpu.make_async_copy(src_hbm.at[pl.ds(0, CHUNK)],
                                      pvs[b], ls[b]).wait()

            def issue_add(b):
                pltpu.async_copy(pvs[b], sh_a.at[idxs[b]], asems[b],
                                 add=True)

            def wait_add(b):
                pltpu.make_async_copy(pvs[b], sh_a.at[idxs[b]],
                                      asems[b]).wait()

            issue_load(ch(0), 0)

            def body(kk, carry):
                c1 = 2 * kk + 1

                @pl.when(kk > 0)
                def _():
                    wait_add(1)

                issue_load(ch(c1), 1)
                wait_load(0)
                issue_add(0)
                wait_add(0)
                issue_load(ch(c1 + 1), 0)
                wait_load(1)
                issue_add(1)
                return carry

            lax.fori_loop(0, (_TILE_BASE - 2) // 2, body, 0)
            wait_add(1)
            issue_load(ch(_TILE_BASE - 1), 1)
            wait_load(0)
            issue_add(0)
            wait_add(0)
            wait_load(1)
            issue_add(1)
            wait_add(1)

            @pl.when(sid < _TILE_EXTRA)
            def _():
                c = _TILE_BASE * SC_TILES + sid
                issue_load(c, 0)
                wait_load(0)
                issue_add(0)
                wait_add(0)

        @pl.when(cid == 0)
        def _():
            run(pa_hbm)

        @pl.when(cid == 1)
        def _():
            run(pb_hbm)

        plsc.subcore_barrier()
        pltpu.sync_copy(sh_a.at[pl.ds(row0, _NPC)],
                        oa_hbm.at[cid, pl.ds(row0, _NPC)])

    return k(pay_a, pay_b, dst, zeros_a)


def _bn_affine(st, rows, g, b):
    s0 = st[0]
    s1 = st[1]
    mean = s0 / rows
    var = s1 / rows - mean * mean
    rstd = lax.rsqrt(var + 1e-5)
    scale = g * rstd
    shift = b - mean * scale
    return scale, shift


def kernel(x, edge_index, edge_attr, params):
    src = edge_index[0]
    dst = edge_index[1]
    e_pad = jnp.pad(edge_attr, ((0, 0), (0, D - EIN)))

    ones = jnp.ones((D,), _f32)
    zeros = jnp.zeros((D,), _f32)
    zeros_a = jnp.zeros((NPAD, D), _f32)

    t_n, sc_n, sh_n = x, ones, zeros
    t_e, sc_e, sh_e = e_pad, ones, zeros

    for i, p in enumerate(params['layers']):
        wkv = jnp.concatenate([p['Wk'], p['Wv']], axis=1)
        kv, q = _kvq(t_n, sc_n, sh_n, wkv, p['Wq'])

        if i == 0:
            we = jnp.pad(p['We'], ((0, D - EIN), (0, 0)))
            ep = _affine_mm(t_e, sc_e, sh_e, we, zeros, BE_ROWS)
            resw = jnp.pad(p['res_e_W'], ((0, D - EIN), (0, 0)))
            eres = _affine_mm(t_e, sc_e, sh_e, resw, p['res_e_b'], BE_ROWS)
            er_t, er_sc, er_sh = eres, ones, zeros
        else:
            ep = _affine_mm(t_e, sc_e, sh_e, p['We'], zeros, BE_ROWS)
            er_t, er_sc, er_sh = t_e, sc_e, sh_e

        gkv, gq = _sc_gather(kv, q, src, dst)
        ye, pa, pb, st_e = _score(gkv, gq, ep, er_t, er_sc, er_sh,
                                  p['Wo_e'])
        acc = _sc_scatter(pa, pb, dst, zeros_a)
        y_n, st_n = _nodeup(acc, t_n, sc_n, sh_n, p['Wo_n'])

        sc1_n, sh1_n = _bn_affine(st_n, N, p['nx_g'], p['nx_b'])
        sc1_e, sh1_e = _bn_affine(st_e, E, p['ne_g'], p['ne_b'])

        z_n, stz_n = _ffn(y_n, sc1_n, sh1_n, p['fn_W1'], p['fn_b1'],
                          p['fn_W2'], p['fn_b2'], BN_ROWS)
        z_e, stz_e = _ffn(ye, sc1_e, sh1_e, p['fe_W1'], p['fe_b1'],
                          p['fe_W2'], p['fe_b2'], BE_ROWS)

        sc_n, sh_n = _bn_affine(stz_n, N, p['fn_g'], p['fn_b'])
        sc_e, sh_e = _bn_affine(stz_e, E, p['fe_g'], p['fe_b'])
        t_n, t_e = z_n, z_e

    npw = jnp.pad(params['np_W'], ((0, 0), (0, D - NC)))
    npb = jnp.pad(params['np_b'], ((0, D - NC),))
    epw = jnp.pad(params['ep_W'], ((0, 0), (0, D - EC)))
    epb = jnp.pad(params['ep_b'], ((0, D - EC),))
    node_pred = _affine_mm(t_n, sc_n, sh_n, npw, npb, BN_ROWS)[:, :NC]
    edge_pred = _affine_mm(t_e, sc_e, sh_e, epw, epb, BE_ROWS)[:, :EC]
    return node_pred, edge_pred, x
